# Initial kernel scaffold; baseline (speedup 1.0000x reference)
import jax
import jax.numpy as jnp
from jax.experimental import pallas as pl


def kernel(x, edge_index, W1, b1, W2, b2):
    raise NotImplementedError("write your pallas kernel here")


# speedup vs baseline: 26.8917x; 26.8917x over previous
import functools

import jax
import jax.numpy as jnp
from jax import lax
from jax.experimental import pallas as pl
from jax.experimental.pallas import tpu as pltpu
from jax.experimental.pallas import tpu_sc as plsc

N = 10000
D = 128
NE = 320000
NC = 2
NS = 16
NW = NC * NS
EB = 128
NB = 80
NE_PAD = NW * NB * EB
PAD = NE_PAD - NE
TRASH = 112
ACC_ROWS = N + TRASH
CH = ACC_ROWS // NS

_MESH = plsc.VectorSubcoreMesh(core_axis_name="c", subcore_axis_name="s")


@functools.partial(
    pl.kernel,
    mesh=_MESH,
    out_type=jax.ShapeDtypeStruct((NC * ACC_ROWS,), jnp.float32),
    scratch_types=[
        pltpu.VMEM((NB, EB), jnp.int32),
        pltpu.VMEM((EB,), jnp.float32),
        pltpu.VMEM((640,), jnp.float32),
        pltpu.VMEM_SHARED((ACC_ROWS,), jnp.float32),
    ],
)
def _sc_hist(dst_hbm, out_hbm, dst_v, ones_v, stage_v, acc):
    c = lax.axis_index("c")
    s = lax.axis_index("s")
    w = c * NS + s
    off = pl.multiple_of(s * CH, 8)
    for i in range(640 // 16):
        stage_v[pl.ds(i * 16, 16)] = jnp.zeros((16,), jnp.float32)
    pltpu.sync_copy(stage_v.at[pl.ds(0, CH)], acc.at[pl.ds(off, CH)])
    for i in range(EB // 16):
        ones_v[pl.ds(i * 16, 16)] = jnp.ones((16,), jnp.float32)
    pltpu.sync_copy(dst_hbm.at[w], dst_v)
    plsc.subcore_barrier()

    def body(j, carry):
        pltpu.sync_copy(ones_v, acc.at[dst_v.at[j]], add=True)
        return carry

    lax.fori_loop(0, NB, body, 0)
    plsc.subcore_barrier()
    out_off = pl.multiple_of(c * ACC_ROWS + s * CH, 8)
    pltpu.sync_copy(acc.at[pl.ds(off, CH)], stage_v.at[pl.ds(0, CH)])
    pltpu.sync_copy(stage_v.at[pl.ds(0, CH)], out_hbm.at[pl.ds(out_off, CH)])


NBC = 16
NCH = NB // NBC


@functools.partial(
    pl.kernel,
    mesh=_MESH,
    out_type=jax.ShapeDtypeStruct((NC, ACC_ROWS, D), jnp.float32),
    scratch_types=[
        pltpu.VMEM((NBC, EB), jnp.int32),
        pltpu.VMEM((NBC, EB), jnp.int32),
        pltpu.VMEM((EB, D), jnp.float32),
        pltpu.VMEM((EB, D), jnp.float32),
        pltpu.VMEM_SHARED((ACC_ROWS, D), jnp.float32),
        pltpu.SemaphoreType.DMA,
        pltpu.SemaphoreType.DMA,
    ],
)
def _sc_scatter(g_hbm, src_hbm, dst_hbm, zeros_hbm, out_hbm,
                src_v, dst_v, rows0, rows1, acc, sem0, sem1):
    c = lax.axis_index("c")
    s = lax.axis_index("s")
    w = c * NS + s
    off = pl.multiple_of(s * CH, 8)
    @pl.when(c == 0)
    def _():
        pltpu.sync_copy(g_hbm.at[pl.ds(off, CH)], acc.at[pl.ds(off, CH)])

    @pl.when(c != 0)
    def _():
        pltpu.sync_copy(zeros_hbm, acc.at[pl.ds(off, CH)])

    plsc.subcore_barrier()

    for k in range(NCH):
        k0 = k * NBC
        pltpu.sync_copy(src_hbm.at[w, pl.ds(k0, NBC)], src_v)
        pltpu.sync_copy(dst_hbm.at[w, pl.ds(k0, NBC)], dst_v)
        pltpu.async_copy(g_hbm.at[src_v.at[0]], rows0, sem0)

        def body(i, carry):
            j0 = i * 2
            pltpu.make_async_copy(g_hbm.at[src_v.at[j0]], rows0, sem0).wait()
            pltpu.async_copy(g_hbm.at[src_v.at[j0 + 1]], rows1, sem1)
            pltpu.sync_copy(rows0, acc.at[dst_v.at[j0]], add=True)

            @pl.when(j0 + 2 < NBC)
            def _():
                pltpu.async_copy(g_hbm.at[src_v.at[j0 + 2]], rows0, sem0)

            pltpu.make_async_copy(g_hbm.at[src_v.at[j0 + 1]], rows1, sem1).wait()
            pltpu.sync_copy(rows1, acc.at[dst_v.at[j0 + 1]], add=True)
            return carry

        lax.fori_loop(0, NBC // 2, body, 0)

    plsc.subcore_barrier()
    pltpu.sync_copy(acc.at[pl.ds(off, CH)], out_hbm.at[c, pl.ds(off, CH)])


_R = 632


def _tc1_body(x_ref, w_ref, d_ref, o_ref):
    dis = lax.rsqrt(d_ref[...] + 1.0)
    o_ref[...] = jnp.dot(x_ref[...], w_ref[...],
                         preferred_element_type=jnp.float32) * dis


def _tc_mid_body(p0_ref, p1_ref, d_ref, b_ref, w_ref, a_ref, g_ref):
    dis = lax.rsqrt(d_ref[...] + 1.0)
    a = (p0_ref[...] + p1_ref[...]) * dis + b_ref[...]
    a_ref[...] = a
    g_ref[...] = jnp.dot(jnp.maximum(a, 0.0), w_ref[...],
                         preferred_element_type=jnp.float32) * dis


_ROWS = pl.BlockSpec((_R, D), lambda i: (i, 0))
_COL = pl.BlockSpec((_R, 1), lambda i: (i, 0))
_WMAT = pl.BlockSpec((D, D), lambda i: (0, 0))
_BIAS = pl.BlockSpec((1, D), lambda i: (0, 0))
_OUTP = jax.ShapeDtypeStruct((ACC_ROWS, D), jnp.float32)

_tc1 = pl.pallas_call(_tc1_body, grid=(ACC_ROWS // _R,),
                      in_specs=[_ROWS, _WMAT, _COL],
                      out_specs=_ROWS, out_shape=_OUTP)
_tc_mid = pl.pallas_call(_tc_mid_body, grid=(ACC_ROWS // _R,),
                         in_specs=[_ROWS, _ROWS, _COL, _BIAS, _WMAT],
                         out_specs=(_ROWS, _ROWS), out_shape=(_OUTP, _OUTP))


def kernel(x, edge_index, W1, b1, W2, b2):
    src = edge_index[0].astype(jnp.int32)
    dst = edge_index[1].astype(jnp.int32)
    pad_ar = jnp.arange(PAD, dtype=jnp.int32)
    src_p = jnp.concatenate([src, pad_ar % N]).reshape(NW, NB, EB)
    dst_p = jnp.concatenate([dst, N + pad_ar % TRASH]).reshape(NW, NB, EB)
    zeros2 = jnp.zeros((CH, D), jnp.float32)
    x_pad = jnp.pad(x, ((0, TRASH), (0, 0)))

    hist = _sc_hist(dst_p).reshape(NC, ACC_ROWS)
    deg = (hist[0] + hist[1]).reshape(ACC_ROWS, 1)

    g1 = _tc1(x_pad, W1, deg)

    ws = jnp.stack([W2, jnp.zeros_like(W2)])
    bs = jnp.stack([b1.reshape(1, D), b2.reshape(1, D)])

    def layer(g, wb):
        w, b = wb
        p = _sc_scatter(g, src_p, dst_p, zeros2)
        a, g_next = _tc_mid(p[0], p[1], deg, b, w)
        return g_next, a

    _, outs = lax.scan(layer, g1, (ws, bs))
    return outs[1, :N]


# speedup vs baseline: 29.1437x; 1.0837x over previous
import functools

import jax
import jax.numpy as jnp
from jax import lax
from jax.experimental import pallas as pl
from jax.experimental.pallas import tpu as pltpu
from jax.experimental.pallas import tpu_sc as plsc

N = 10000
D = 128
NE = 320000
NC = 2
NS = 16
NW = NC * NS
EB = 128
NB = 80
NE_PAD = NW * NB * EB
PAD = NE_PAD - NE
TRASH = 112
ACC_ROWS = N + TRASH
CH = ACC_ROWS // NS

_MESH = plsc.VectorSubcoreMesh(core_axis_name="c", subcore_axis_name="s")


@functools.partial(
    pl.kernel,
    mesh=_MESH,
    out_type=jax.ShapeDtypeStruct((NC * ACC_ROWS,), jnp.float32),
    scratch_types=[
        pltpu.VMEM((NB, EB), jnp.int32),
        pltpu.VMEM((EB,), jnp.float32),
        pltpu.VMEM((640,), jnp.float32),
        pltpu.VMEM_SHARED((ACC_ROWS,), jnp.float32),
    ],
)
def _sc_hist(dst_hbm, out_hbm, dst_v, ones_v, stage_v, acc):
    c = lax.axis_index("c")
    s = lax.axis_index("s")
    w = c * NS + s
    off = pl.multiple_of(s * CH, 8)
    for i in range(640 // 16):
        stage_v[pl.ds(i * 16, 16)] = jnp.zeros((16,), jnp.float32)
    pltpu.sync_copy(stage_v.at[pl.ds(0, CH)], acc.at[pl.ds(off, CH)])
    for i in range(EB // 16):
        ones_v[pl.ds(i * 16, 16)] = jnp.ones((16,), jnp.float32)
    pltpu.sync_copy(dst_hbm.at[w], dst_v)
    plsc.subcore_barrier()

    def body(j, carry):
        pltpu.sync_copy(ones_v, acc.at[dst_v.at[j]], add=True)
        return carry

    lax.fori_loop(0, NB, body, 0)
    plsc.subcore_barrier()
    out_off = pl.multiple_of(c * ACC_ROWS + s * CH, 8)
    pltpu.sync_copy(acc.at[pl.ds(off, CH)], stage_v.at[pl.ds(0, CH)])
    pltpu.sync_copy(stage_v.at[pl.ds(0, CH)], out_hbm.at[pl.ds(out_off, CH)])


NBC = 40
NCH = NB // NBC


@functools.partial(
    pl.kernel,
    mesh=_MESH,
    out_type=jax.ShapeDtypeStruct((NC, ACC_ROWS, D), jnp.float32),
    scratch_types=[
        pltpu.VMEM((NBC, EB), jnp.int32),
        pltpu.VMEM((NBC, EB), jnp.int32),
        pltpu.VMEM((EB, D), jnp.float32),
        pltpu.VMEM((EB, D), jnp.float32),
        pltpu.VMEM_SHARED((ACC_ROWS, D), jnp.float32),
        pltpu.SemaphoreType.DMA,
        pltpu.SemaphoreType.DMA,
    ],
)
def _sc_scatter(g_hbm, src_hbm, dst_hbm, zeros_hbm, out_hbm,
                src_v, dst_v, rows0, rows1, acc, sem0, sem1):
    c = lax.axis_index("c")
    s = lax.axis_index("s")
    w = c * NS + s
    off = pl.multiple_of(s * CH, 8)
    @pl.when(c == 0)
    def _():
        pltpu.sync_copy(g_hbm.at[pl.ds(off, CH)], acc.at[pl.ds(off, CH)])

    @pl.when(c != 0)
    def _():
        pltpu.sync_copy(zeros_hbm, acc.at[pl.ds(off, CH)])

    plsc.subcore_barrier()

    for k in range(NCH):
        k0 = k * NBC
        pltpu.sync_copy(src_hbm.at[w, pl.ds(k0, NBC)], src_v)
        pltpu.sync_copy(dst_hbm.at[w, pl.ds(k0, NBC)], dst_v)
        pltpu.async_copy(g_hbm.at[src_v.at[0]], rows0, sem0)

        def body(i, carry):
            j0 = i * 2
            pltpu.async_copy(g_hbm.at[src_v.at[j0 + 1]], rows1, sem1)
            pltpu.make_async_copy(g_hbm.at[src_v.at[j0]], rows0, sem0).wait()
            pltpu.sync_copy(rows0, acc.at[dst_v.at[j0]], add=True)

            @pl.when(j0 + 2 < NBC)
            def _():
                pltpu.async_copy(g_hbm.at[src_v.at[j0 + 2]], rows0, sem0)

            pltpu.make_async_copy(g_hbm.at[src_v.at[j0 + 1]], rows1, sem1).wait()
            pltpu.sync_copy(rows1, acc.at[dst_v.at[j0 + 1]], add=True)
            return carry

        lax.fori_loop(0, NBC // 2, body, 0)

    plsc.subcore_barrier()
    pltpu.sync_copy(acc.at[pl.ds(off, CH)], out_hbm.at[c, pl.ds(off, CH)])


_R = 632


def _tc1_body(x_ref, w_ref, d_ref, o_ref):
    dis = lax.rsqrt(d_ref[...] + 1.0)
    o_ref[...] = jnp.dot(x_ref[...], w_ref[...],
                         preferred_element_type=jnp.float32) * dis


def _tc_mid_body(p0_ref, p1_ref, d_ref, b_ref, w_ref, a_ref, g_ref):
    dis = lax.rsqrt(d_ref[...] + 1.0)
    a = (p0_ref[...] + p1_ref[...]) * dis + b_ref[...]
    a_ref[...] = a
    g_ref[...] = jnp.dot(jnp.maximum(a, 0.0), w_ref[...],
                         preferred_element_type=jnp.float32) * dis


_ROWS = pl.BlockSpec((_R, D), lambda i: (i, 0))
_COL = pl.BlockSpec((_R, 1), lambda i: (i, 0))
_WMAT = pl.BlockSpec((D, D), lambda i: (0, 0))
_BIAS = pl.BlockSpec((1, D), lambda i: (0, 0))
_OUTP = jax.ShapeDtypeStruct((ACC_ROWS, D), jnp.float32)

_tc1 = pl.pallas_call(_tc1_body, grid=(ACC_ROWS // _R,),
                      in_specs=[_ROWS, _WMAT, _COL],
                      out_specs=_ROWS, out_shape=_OUTP)
_tc_mid = pl.pallas_call(_tc_mid_body, grid=(ACC_ROWS // _R,),
                         in_specs=[_ROWS, _ROWS, _COL, _BIAS, _WMAT],
                         out_specs=(_ROWS, _ROWS), out_shape=(_OUTP, _OUTP))


def kernel(x, edge_index, W1, b1, W2, b2):
    src = edge_index[0].astype(jnp.int32)
    dst = edge_index[1].astype(jnp.int32)
    pad_ar = jnp.arange(PAD, dtype=jnp.int32)
    src_p = jnp.concatenate([src, pad_ar % N]).reshape(NW, NB, EB)
    dst_p = jnp.concatenate([dst, N + pad_ar % TRASH]).reshape(NW, NB, EB)
    zeros2 = jnp.zeros((CH, D), jnp.float32)
    x_pad = jnp.pad(x, ((0, TRASH), (0, 0)))

    hist = _sc_hist(dst_p).reshape(NC, ACC_ROWS)
    deg = (hist[0] + hist[1]).reshape(ACC_ROWS, 1)

    g1 = _tc1(x_pad, W1, deg)

    ws = jnp.stack([W2, jnp.zeros_like(W2)])
    bs = jnp.stack([b1.reshape(1, D), b2.reshape(1, D)])

    def layer(g, wb):
        w, b = wb
        p = _sc_scatter(g, src_p, dst_p, zeros2)
        a, g_next = _tc_mid(p[0], p[1], deg, b, w)
        return g_next, a

    _, outs = lax.scan(layer, g1, (ws, bs))
    return outs[1, :N]
